# Initial kernel scaffold; baseline (speedup 1.0000x reference)
import jax
import jax.numpy as jnp
from jax.experimental import pallas as pl


def kernel(x, W_q, keys1, keys2, values):
    raise NotImplementedError("write your pallas kernel here")


# speedup vs baseline: 1.9950x; 1.9950x over previous
import numpy as np

import jax
import jax.numpy as jnp
from jax import lax
from jax.experimental import pallas as pl
from jax.experimental.pallas import tpu as pltpu
from jax.experimental.pallas import tpu_sc as plsc

_B = 4096
_INPUT_DIM = 512
_OUT_DIM = 512
_K_DIM = 256
_HALF = _K_DIM // 2
_HEADS = 4
_KNN = 32
_N_KEYS = 512

_BB = 256
_NCAND = 128
_BPW = _B // 32

_NEGF = -3.0e38
_BIGI = 1 << 20

_HP = lax.Precision.HIGHEST


def _build_candidates():
    pairs = [(i, j) for i in range(_KNN) for j in range(_KNN)
             if (i + 1) * (j + 1) <= _KNN]
    fc = np.full((1, _NCAND), _BIGI, np.int32)
    cb = np.zeros((1, _NCAND), np.float32)
    for c, (i, j) in enumerate(pairs):
        fc[0, c] = i * _KNN + j
    cb[0, len(pairs):] = _NEGF
    return fc, cb


_FC, _CB = _build_candidates()


def _bf16_dot(a, b):
    return lax.dot_general(a.astype(jnp.bfloat16), b.astype(jnp.bfloat16),
                           (((1,), (1,)), ((), ())),
                           preferred_element_type=jnp.float32)


def _topk32(s):
    iota = lax.broadcasted_iota(jnp.int32, s.shape, 1)
    vals, idxs = [], []
    cur = s
    for _ in range(_KNN):
        m = jnp.max(cur, axis=1, keepdims=True)
        ix = jnp.min(jnp.where(cur == m, iota, _BIGI), axis=1, keepdims=True)
        vals.append(m)
        idxs.append(ix)
        cur = jnp.where(iota == ix, _NEGF, cur)
    return jnp.concatenate(vals, 1), jnp.concatenate(idxs, 1)


def _topk32_cand(cand, fcb):
    vals, codes = [], []
    cur = cand
    for _ in range(_KNN):
        m = jnp.max(cur, axis=1, keepdims=True)
        code = jnp.min(jnp.where(cur == m, fcb, _BIGI), axis=1, keepdims=True)
        vals.append(m)
        codes.append(code)
        cur = jnp.where(fcb == code, _NEGF, cur)
    return jnp.concatenate(vals, 1), jnp.concatenate(codes, 1)


def _gather_pos(tab, pos):
    ii = lax.broadcasted_iota(jnp.int32, (_BB, _KNN, _KNN), 2)
    onehot = pos[:, :, None] == ii
    return jnp.sum(jnp.where(onehot, tab[:, None, :], 0), axis=2)


def _pair_gather(sc, pos_const, io3):
    onehot = pos_const == io3
    return jnp.sum(jnp.where(onehot, sc[:, None, :], 0.0), axis=2)


def _tc_body(x_ref, wq_ref, k1_ref, k2_ref, fc_ref, cb_ref,
             idx_ref, w_ref):
    x = x_ref[...]
    q = _bf16_dot(x, wq_ref[...])
    fcb = jnp.broadcast_to(fc_ref[...], (_BB, _NCAND))
    cb = jnp.broadcast_to(cb_ref[...], (_BB, _NCAND))
    io3 = lax.broadcasted_iota(jnp.int32, (_BB, _NCAND, _KNN), 2)
    ic = jnp.broadcast_to(lax.shift_right_logical(fc_ref[...], 5)[..., None],
                          (_BB, _NCAND, 1))
    jc = jnp.broadcast_to(lax.bitwise_and(fc_ref[...], _KNN - 1)[..., None],
                          (_BB, _NCAND, 1))
    for h in range(_HEADS):
        q1 = q[:, h * _K_DIM: h * _K_DIM + _HALF]
        q2 = q[:, h * _K_DIM + _HALF: (h + 1) * _K_DIM]
        s1 = _bf16_dot(q1, k1_ref[h])
        s2 = _bf16_dot(q2, k2_ref[h])
        sc1, i1 = _topk32(s1)
        sc2, i2 = _topk32(s2)
        c1 = _pair_gather(sc1, ic, io3)
        c2 = _pair_gather(sc2, jc, io3)
        scores, code = _topk32_cand(c1 + c2 + cb, fcb)
        ipos = lax.shift_right_logical(code, 5)
        jpos = lax.bitwise_and(code, _KNN - 1)
        g1 = _gather_pos(i1, ipos)
        g2 = _gather_pos(i2, jpos)
        e = jnp.exp(scores - scores[:, 0:1])
        w = e / jnp.sum(e, axis=1, keepdims=True)
        idx_ref[:, h * _KNN:(h + 1) * _KNN] = g1 * _N_KEYS + g2
        w_ref[:, h * _KNN:(h + 1) * _KNN] = w


def _tc_call(x, W_q, keys1, keys2):
    nsel = _HEADS * _KNN
    return pl.pallas_call(
        _tc_body,
        grid=(_B // _BB,),
        in_specs=[
            pl.BlockSpec((_BB, _INPUT_DIM), lambda i: (i, 0)),
            pl.BlockSpec((_HEADS * _K_DIM, _INPUT_DIM), lambda i: (0, 0)),
            pl.BlockSpec((_HEADS, _N_KEYS, _HALF), lambda i: (0, 0, 0)),
            pl.BlockSpec((_HEADS, _N_KEYS, _HALF), lambda i: (0, 0, 0)),
            pl.BlockSpec((1, _NCAND), lambda i: (0, 0)),
            pl.BlockSpec((1, _NCAND), lambda i: (0, 0)),
        ],
        out_specs=[
            pl.BlockSpec((_BB, nsel), lambda i: (i, 0)),
            pl.BlockSpec((_BB, nsel), lambda i: (i, 0)),
        ],
        out_shape=[
            jax.ShapeDtypeStruct((_B, nsel), jnp.int32),
            jax.ShapeDtypeStruct((_B, nsel), jnp.float32),
        ],
    )(x, W_q, keys1, keys2, jnp.asarray(_FC), jnp.asarray(_CB))


def _sc_body(vals_hbm, idx_hbm, w_hbm, out_hbm, idx_v, w_v, rows_v, acc_v,
             sem):
    nsel = _HEADS * _KNN
    wid = lax.axis_index("s") * 2 + lax.axis_index("c")
    base = wid * _BPW
    pltpu.sync_copy(idx_hbm.at[pl.ds(base, _BPW)], idx_v)
    pltpu.sync_copy(w_hbm.at[pl.ds(base, _BPW)], w_v)

    def b_body(bb, carry):
        for c in range(_OUT_DIM // 16):
            acc_v[pl.ds(c * 16, 16)] = jnp.zeros((16,), jnp.float32)
        pltpu.async_copy(vals_hbm.at[idx_v.at[bb]], rows_v, sem).wait()

        def g_body(g, carry2):
            wv16 = w_v[bb, pl.ds(g * 16, 16)]
            for jj in range(16):
                wvec = jnp.full((16,), wv16[jj], jnp.float32)
                j = g * 16 + jj
                for c in range(_OUT_DIM // 16):
                    plsc.addupdate(acc_v.at[pl.ds(c * 16, 16)],
                                   wvec * rows_v[j, pl.ds(c * 16, 16)])
            return carry2

        lax.fori_loop(0, nsel // 16, g_body, 0)
        pltpu.sync_copy(acc_v, out_hbm.at[base + bb])
        return carry

    lax.fori_loop(0, _BPW, b_body, 0)


def _sc_call(values, idx, w):
    nsel = _HEADS * _KNN
    fn = pl.kernel(
        _sc_body,
        out_type=jax.ShapeDtypeStruct((_B, _OUT_DIM), jnp.float32),
        mesh=plsc.VectorSubcoreMesh(core_axis_name="c", subcore_axis_name="s",
                                    num_cores=2, num_subcores=16),
        scratch_types=[
            pltpu.VMEM((_BPW, nsel), jnp.int32),
            pltpu.VMEM((_BPW, nsel), jnp.float32),
            pltpu.VMEM((nsel, _OUT_DIM), jnp.float32),
            pltpu.VMEM((_OUT_DIM,), jnp.float32),
            pltpu.SemaphoreType.DMA,
        ],
    )
    return fn(values, idx, w)


def kernel(x, W_q, keys1, keys2, values):
    idx, w = _tc_call(x, W_q, keys1, keys2)
    return _sc_call(values, idx, w)


# speedup vs baseline: 2.0287x; 1.0169x over previous
import numpy as np

import jax
import jax.numpy as jnp
from jax import lax
from jax.experimental import pallas as pl
from jax.experimental.pallas import tpu as pltpu
from jax.experimental.pallas import tpu_sc as plsc

_B = 4096
_INPUT_DIM = 512
_OUT_DIM = 512
_K_DIM = 256
_HALF = _K_DIM // 2
_HEADS = 4
_KNN = 32
_N_KEYS = 512

_BB = 256
_NCAND = 128
_BPW = _B // 32

_NEGF = -3.0e38
_BIGI = 1 << 20

_HP = lax.Precision.HIGHEST


def _build_candidates():
    pairs = [(i, j) for i in range(_KNN) for j in range(_KNN)
             if (i + 1) * (j + 1) <= _KNN]
    fc = np.full((1, _NCAND), _BIGI, np.int32)
    cb = np.zeros((1, _NCAND), np.float32)
    for c, (i, j) in enumerate(pairs):
        fc[0, c] = i * _KNN + j
    cb[0, len(pairs):] = _NEGF
    return fc, cb


_FC, _CB = _build_candidates()


def _bf16_dot(a, b):
    return lax.dot_general(a.astype(jnp.bfloat16), b.astype(jnp.bfloat16),
                           (((1,), (1,)), ((), ())),
                           preferred_element_type=jnp.float32)


def _topk32(s):
    iota = lax.broadcasted_iota(jnp.int32, s.shape, 1)
    vals, idxs = [], []
    cur = s
    for _ in range(_KNN):
        m = jnp.max(cur, axis=1, keepdims=True)
        ix = jnp.min(jnp.where(cur == m, iota, _BIGI), axis=1, keepdims=True)
        vals.append(m)
        idxs.append(ix)
        cur = jnp.where(iota == ix, _NEGF, cur)
    return jnp.concatenate(vals, 1), jnp.concatenate(idxs, 1)


def _topk32_cand(cand, fcb):
    vals, codes = [], []
    cur = cand
    for _ in range(_KNN):
        m = jnp.max(cur, axis=1, keepdims=True)
        code = jnp.min(jnp.where(cur == m, fcb, _BIGI), axis=1, keepdims=True)
        vals.append(m)
        codes.append(code)
        cur = jnp.where(fcb == code, _NEGF, cur)
    return jnp.concatenate(vals, 1), jnp.concatenate(codes, 1)


def _gather_pos(tab, pos):
    ii = lax.broadcasted_iota(jnp.int32, (_BB, _KNN, _KNN), 2)
    onehot = pos[:, :, None] == ii
    return jnp.sum(jnp.where(onehot, tab[:, None, :], 0), axis=2)


def _pair_gather(sc, pos_const, io3):
    onehot = pos_const == io3
    return jnp.sum(jnp.where(onehot, sc[:, None, :], 0.0), axis=2)


def _tc_body(x_ref, wq_ref, k1_ref, k2_ref, fc_ref, cb_ref,
             idx_ref, w_ref):
    x = x_ref[...]
    q = _bf16_dot(x, wq_ref[...])
    fcb = jnp.broadcast_to(fc_ref[...], (_BB, _NCAND))
    cb = jnp.broadcast_to(cb_ref[...], (_BB, _NCAND))
    io3 = lax.broadcasted_iota(jnp.int32, (_BB, _NCAND, _KNN), 2)
    ic = jnp.broadcast_to(lax.shift_right_logical(fc_ref[...], 5)[..., None],
                          (_BB, _NCAND, 1))
    jc = jnp.broadcast_to(lax.bitwise_and(fc_ref[...], _KNN - 1)[..., None],
                          (_BB, _NCAND, 1))
    for h in range(_HEADS):
        q1 = q[:, h * _K_DIM: h * _K_DIM + _HALF]
        q2 = q[:, h * _K_DIM + _HALF: (h + 1) * _K_DIM]
        s1 = _bf16_dot(q1, k1_ref[h])
        s2 = _bf16_dot(q2, k2_ref[h])
        sc1, i1 = _topk32(s1)
        sc2, i2 = _topk32(s2)
        c1 = _pair_gather(sc1, ic, io3)
        c2 = _pair_gather(sc2, jc, io3)
        scores, code = _topk32_cand(c1 + c2 + cb, fcb)
        ipos = lax.shift_right_logical(code, 5)
        jpos = lax.bitwise_and(code, _KNN - 1)
        g1 = _gather_pos(i1, ipos)
        g2 = _gather_pos(i2, jpos)
        e = jnp.exp(scores - scores[:, 0:1])
        w = e / jnp.sum(e, axis=1, keepdims=True)
        idx_ref[:, h * _KNN:(h + 1) * _KNN] = g1 * _N_KEYS + g2
        w_ref[:, h * _KNN:(h + 1) * _KNN] = w


def _tc_call(x, W_q, keys1, keys2):
    nsel = _HEADS * _KNN
    return pl.pallas_call(
        _tc_body,
        grid=(_B // _BB,),
        in_specs=[
            pl.BlockSpec((_BB, _INPUT_DIM), lambda i: (i, 0)),
            pl.BlockSpec((_HEADS * _K_DIM, _INPUT_DIM), lambda i: (0, 0)),
            pl.BlockSpec((_HEADS, _N_KEYS, _HALF), lambda i: (0, 0, 0)),
            pl.BlockSpec((_HEADS, _N_KEYS, _HALF), lambda i: (0, 0, 0)),
            pl.BlockSpec((1, _NCAND), lambda i: (0, 0)),
            pl.BlockSpec((1, _NCAND), lambda i: (0, 0)),
        ],
        out_specs=[
            pl.BlockSpec((_BB, nsel), lambda i: (i, 0)),
            pl.BlockSpec((_BB, nsel), lambda i: (i, 0)),
        ],
        out_shape=[
            jax.ShapeDtypeStruct((_B, nsel), jnp.int32),
            jax.ShapeDtypeStruct((_B, nsel), jnp.float32),
        ],
    )(x, W_q, keys1, keys2, jnp.asarray(_FC), jnp.asarray(_CB))


def _sc_body(vals_hbm, idx_hbm, w_hbm, out_hbm, idx_v, w_v, rows_v, acc_v,
             sems):
    nch = 2 * _BPW
    wid = lax.axis_index("s") * 2 + lax.axis_index("c")
    base = wid * _BPW
    pltpu.sync_copy(idx_hbm.at[pl.ds(base, _BPW)], idx_v)
    pltpu.sync_copy(w_hbm.at[pl.ds(base, _BPW)], w_v)

    def copy_for(t, slot):
        b = lax.shift_right_logical(t, 1)
        half = lax.bitwise_and(t, 1)
        return pltpu.make_async_copy(
            vals_hbm.at[idx_v.at[b, pl.ds(half * 64, 64)]],
            rows_v.at[slot], sems.at[slot])

    copy_for(0, 0).start()

    def t_body(t, carry):
        b = lax.shift_right_logical(t, 1)
        half = lax.bitwise_and(t, 1)
        slot = lax.bitwise_and(t, 1)

        @pl.when(half == 0)
        def _zero():
            for c in range(_OUT_DIM // 16):
                acc_v[pl.ds(c * 16, 16)] = jnp.zeros((16,), jnp.float32)

        @pl.when(t + 1 < nch)
        def _issue():
            copy_for(t + 1, lax.bitwise_and(t + 1, 1)).start()

        copy_for(t, slot).wait()

        def g_body(g, carry2):
            wv16 = w_v[b, pl.ds(half * 64 + g * 16, 16)]
            for jj in range(16):
                wvec = jnp.full((16,), wv16[jj], jnp.float32)
                for c in range(_OUT_DIM // 16):
                    plsc.addupdate(
                        acc_v.at[pl.ds(c * 16, 16)],
                        wvec * rows_v[slot, g * 16 + jj, pl.ds(c * 16, 16)])
            return carry2

        lax.fori_loop(0, 4, g_body, 0)

        @pl.when(half == 1)
        def _flush():
            pltpu.sync_copy(acc_v, out_hbm.at[base + b])

        return carry

    lax.fori_loop(0, nch, t_body, 0)


def _sc_call(values, idx, w):
    nsel = _HEADS * _KNN
    fn = pl.kernel(
        _sc_body,
        out_type=jax.ShapeDtypeStruct((_B, _OUT_DIM), jnp.float32),
        mesh=plsc.VectorSubcoreMesh(core_axis_name="c", subcore_axis_name="s",
                                    num_cores=2, num_subcores=16),
        scratch_types=[
            pltpu.VMEM((_BPW, nsel), jnp.int32),
            pltpu.VMEM((_BPW, nsel), jnp.float32),
            pltpu.VMEM((2, nsel // 2, _OUT_DIM), jnp.float32),
            pltpu.VMEM((_OUT_DIM,), jnp.float32),
            pltpu.SemaphoreType.DMA((2,)),
        ],
    )
    return fn(values, idx, w)


def kernel(x, W_q, keys1, keys2, values):
    idx, w = _tc_call(x, W_q, keys1, keys2)
    return _sc_call(values, idx, w)


# speedup vs baseline: 2.9639x; 1.4610x over previous
import numpy as np

import jax
import jax.numpy as jnp
from jax import lax
from jax.experimental import pallas as pl
from jax.experimental.pallas import tpu as pltpu
from jax.experimental.pallas import tpu_sc as plsc

_B = 4096
_INPUT_DIM = 512
_OUT_DIM = 512
_K_DIM = 256
_HALF = _K_DIM // 2
_HEADS = 4
_KNN = 32
_N_KEYS = 512

_BB = 256
_NCAND = 128
_BPW = _B // 32

_NEGF = -3.0e38
_BIGI = 1 << 20

_HP = lax.Precision.HIGHEST


def _build_candidates():
    pairs = [(i, j) for i in range(_KNN) for j in range(_KNN)
             if (i + 1) * (j + 1) <= _KNN]
    fc = np.full((1, _NCAND), _BIGI, np.int32)
    cb = np.zeros((1, _NCAND), np.float32)
    for c, (i, j) in enumerate(pairs):
        fc[0, c] = i * _KNN + j
    cb[0, len(pairs):] = _NEGF
    return fc, cb


_FC, _CB = _build_candidates()


def _bf16_dot(a, b):
    return lax.dot_general(a.astype(jnp.bfloat16), b.astype(jnp.bfloat16),
                           (((1,), (1,)), ((), ())),
                           preferred_element_type=jnp.float32)


def _topk32(s):
    iota = lax.broadcasted_iota(jnp.int32, s.shape, 1)
    vals, idxs = [], []
    cur = s
    for _ in range(_KNN):
        m = jnp.max(cur, axis=1, keepdims=True)
        ix = jnp.min(jnp.where(cur == m, iota, _BIGI), axis=1, keepdims=True)
        vals.append(m)
        idxs.append(ix)
        cur = jnp.where(iota == ix, _NEGF, cur)
    return jnp.concatenate(vals, 1), jnp.concatenate(idxs, 1)


def _topk32_cand(cand, fcb):
    vals, codes = [], []
    cur = cand
    for _ in range(_KNN):
        m = jnp.max(cur, axis=1, keepdims=True)
        code = jnp.min(jnp.where(cur == m, fcb, _BIGI), axis=1, keepdims=True)
        vals.append(m)
        codes.append(code)
        cur = jnp.where(fcb == code, _NEGF, cur)
    return jnp.concatenate(vals, 1), jnp.concatenate(codes, 1)


def _gather_pos(tab, pos):
    ii = lax.broadcasted_iota(jnp.int32, (_BB, _KNN, _KNN), 2)
    onehot = pos[:, :, None] == ii
    return jnp.sum(jnp.where(onehot, tab[:, None, :], 0), axis=2)


def _pair_gather(sc, pos_const, io3):
    onehot = pos_const == io3
    return jnp.sum(jnp.where(onehot, sc[:, None, :], 0.0), axis=2)


def _tc_body(x_ref, wq_ref, k1_ref, k2_ref, fc_ref, cb_ref,
             idx_ref, w_ref):
    x = x_ref[...]
    q = _bf16_dot(x, wq_ref[...])
    fcb = jnp.broadcast_to(fc_ref[...], (_BB, _NCAND))
    cb = jnp.broadcast_to(cb_ref[...], (_BB, _NCAND))
    io3 = lax.broadcasted_iota(jnp.int32, (_BB, _NCAND, _KNN), 2)
    ic = jnp.broadcast_to(lax.shift_right_logical(fc_ref[...], 5)[..., None],
                          (_BB, _NCAND, 1))
    jc = jnp.broadcast_to(lax.bitwise_and(fc_ref[...], _KNN - 1)[..., None],
                          (_BB, _NCAND, 1))
    for h in range(_HEADS):
        q1 = q[:, h * _K_DIM: h * _K_DIM + _HALF]
        q2 = q[:, h * _K_DIM + _HALF: (h + 1) * _K_DIM]
        s1 = _bf16_dot(q1, k1_ref[h])
        s2 = _bf16_dot(q2, k2_ref[h])
        sc1, i1 = _topk32(s1)
        sc2, i2 = _topk32(s2)
        c1 = _pair_gather(sc1, ic, io3)
        c2 = _pair_gather(sc2, jc, io3)
        scores, code = _topk32_cand(c1 + c2 + cb, fcb)
        ipos = lax.shift_right_logical(code, 5)
        jpos = lax.bitwise_and(code, _KNN - 1)
        g1 = _gather_pos(i1, ipos)
        g2 = _gather_pos(i2, jpos)
        e = jnp.exp(scores - scores[:, 0:1])
        w = e / jnp.sum(e, axis=1, keepdims=True)
        idx_ref[:, h * _KNN:(h + 1) * _KNN] = g1 * _N_KEYS + g2
        w_ref[:, h * _KNN:(h + 1) * _KNN] = w


def _tc_call(x, W_q, keys1, keys2):
    nsel = _HEADS * _KNN
    return pl.pallas_call(
        _tc_body,
        grid=(_B // _BB,),
        in_specs=[
            pl.BlockSpec((_BB, _INPUT_DIM), lambda i: (i, 0)),
            pl.BlockSpec((_HEADS * _K_DIM, _INPUT_DIM), lambda i: (0, 0)),
            pl.BlockSpec((_HEADS, _N_KEYS, _HALF), lambda i: (0, 0, 0)),
            pl.BlockSpec((_HEADS, _N_KEYS, _HALF), lambda i: (0, 0, 0)),
            pl.BlockSpec((1, _NCAND), lambda i: (0, 0)),
            pl.BlockSpec((1, _NCAND), lambda i: (0, 0)),
        ],
        out_specs=[
            pl.BlockSpec((_BB, nsel), lambda i: (i, 0)),
            pl.BlockSpec((_BB, nsel), lambda i: (i, 0)),
        ],
        out_shape=[
            jax.ShapeDtypeStruct((_B, nsel), jnp.int32),
            jax.ShapeDtypeStruct((_B, nsel), jnp.float32),
        ],
    )(x, W_q, keys1, keys2, jnp.asarray(_FC), jnp.asarray(_CB))


def _sc_body(vals_hbm, idx_hbm, w_hbm, out_hbm, idx_v, w_v, rows_v, acc_v,
             sems):
    nch = 2 * _BPW
    wid = lax.axis_index("s") * 2 + lax.axis_index("c")
    base = wid * _BPW
    pltpu.sync_copy(idx_hbm.at[pl.ds(base, _BPW)], idx_v)
    pltpu.sync_copy(w_hbm.at[pl.ds(base, _BPW)], w_v)

    def copy_for(t, slot):
        b = lax.shift_right_logical(t, 1)
        half = lax.bitwise_and(t, 1)
        return pltpu.make_async_copy(
            vals_hbm.at[idx_v.at[b, pl.ds(half * 64, 64)]],
            rows_v.at[slot], sems.at[slot])

    copy_for(0, 0).start()
    nacc = _OUT_DIM // 16

    def t_body(t, acc):
        b = lax.shift_right_logical(t, 1)
        half = lax.bitwise_and(t, 1)
        slot = lax.bitwise_and(t, 1)
        zero = half == 0
        acc = tuple(jnp.where(zero, 0.0, a) for a in acc)

        @pl.when(t + 1 < nch)
        def _issue():
            copy_for(t + 1, lax.bitwise_and(t + 1, 1)).start()

        copy_for(t, slot).wait()

        def g_body(g, acc2):
            acc2 = list(acc2)
            wv16 = w_v[b, pl.ds(half * 64 + g * 16, 16)]
            for jj in range(16):
                wvec = jnp.full((16,), wv16[jj], jnp.float32)
                for c in range(nacc):
                    acc2[c] = acc2[c] + wvec * rows_v[
                        slot, g * 16 + jj, pl.ds(c * 16, 16)]
            return tuple(acc2)

        acc = lax.fori_loop(0, 4, g_body, acc)

        @pl.when(half == 1)
        def _flush():
            for c in range(nacc):
                acc_v[pl.ds(c * 16, 16)] = acc[c]
            pltpu.sync_copy(acc_v, out_hbm.at[base + b])

        return acc

    zeros = tuple(jnp.zeros((16,), jnp.float32) for _ in range(nacc))
    lax.fori_loop(0, nch, t_body, zeros)


def _sc_call(values, idx, w):
    nsel = _HEADS * _KNN
    fn = pl.kernel(
        _sc_body,
        out_type=jax.ShapeDtypeStruct((_B, _OUT_DIM), jnp.float32),
        mesh=plsc.VectorSubcoreMesh(core_axis_name="c", subcore_axis_name="s",
                                    num_cores=2, num_subcores=16),
        scratch_types=[
            pltpu.VMEM((_BPW, nsel), jnp.int32),
            pltpu.VMEM((_BPW, nsel), jnp.float32),
            pltpu.VMEM((2, nsel // 2, _OUT_DIM), jnp.float32),
            pltpu.VMEM((_OUT_DIM,), jnp.float32),
            pltpu.SemaphoreType.DMA((2,)),
        ],
    )
    return fn(values, idx, w)


def kernel(x, W_q, keys1, keys2, values):
    idx, w = _tc_call(x, W_q, keys1, keys2)
    return _sc_call(values, idx, w)


# speedup vs baseline: 3.5318x; 1.1916x over previous
import numpy as np

import jax
import jax.numpy as jnp
from jax import lax
from jax.experimental import pallas as pl
from jax.experimental.pallas import tpu as pltpu
from jax.experimental.pallas import tpu_sc as plsc

_B = 4096
_INPUT_DIM = 512
_OUT_DIM = 512
_K_DIM = 256
_HALF = _K_DIM // 2
_HEADS = 4
_KNN = 32
_N_KEYS = 512

_BB = 256
_NCAND = 128
_BPW = _B // 32

_NEGF = -3.0e38
_BIGI = 1 << 20

_HP = lax.Precision.HIGHEST


def _build_candidates():
    pairs = [(i, j) for i in range(_KNN) for j in range(_KNN)
             if (i + 1) * (j + 1) <= _KNN]
    fc = np.full((1, _NCAND), _BIGI, np.int32)
    cb = np.zeros((1, _NCAND), np.float32)
    for c, (i, j) in enumerate(pairs):
        fc[0, c] = i * _KNN + j
    cb[0, len(pairs):] = _NEGF
    return fc, cb


_FC, _CB = _build_candidates()


def _bf16_dot(a, b):
    return lax.dot_general(a.astype(jnp.bfloat16), b.astype(jnp.bfloat16),
                           (((1,), (1,)), ((), ())),
                           preferred_element_type=jnp.float32)


def _topk32(s):
    iota = lax.broadcasted_iota(jnp.int32, s.shape, 1)
    vals, idxs = [], []
    cur = s
    for _ in range(_KNN):
        m = jnp.max(cur, axis=1, keepdims=True)
        ix = jnp.min(jnp.where(cur == m, iota, _BIGI), axis=1, keepdims=True)
        vals.append(m)
        idxs.append(ix)
        cur = jnp.where(iota == ix, _NEGF, cur)
    return jnp.concatenate(vals, 1), jnp.concatenate(idxs, 1)


def _topk32_cand(cand, fcb):
    vals, codes = [], []
    cur = cand
    for _ in range(_KNN):
        m = jnp.max(cur, axis=1, keepdims=True)
        code = jnp.min(jnp.where(cur == m, fcb, _BIGI), axis=1, keepdims=True)
        vals.append(m)
        codes.append(code)
        cur = jnp.where(fcb == code, _NEGF, cur)
    return jnp.concatenate(vals, 1), jnp.concatenate(codes, 1)


def _gather_pos(tab, pos):
    ii = lax.broadcasted_iota(jnp.int32, (_BB, _KNN, _KNN), 2)
    onehot = pos[:, :, None] == ii
    return jnp.sum(jnp.where(onehot, tab[:, None, :], 0), axis=2)


def _pair_gather(sc, pos_const, io3):
    onehot = pos_const == io3
    return jnp.sum(jnp.where(onehot, sc[:, None, :], 0.0), axis=2)


def _tc_body(x_ref, wq_ref, k1_ref, k2_ref, fc_ref, cb_ref,
             idx_ref, w_ref):
    x = x_ref[...]
    q = _bf16_dot(x, wq_ref[...])
    fcb = jnp.broadcast_to(fc_ref[...], (_BB, _NCAND))
    cb = jnp.broadcast_to(cb_ref[...], (_BB, _NCAND))
    io3 = lax.broadcasted_iota(jnp.int32, (_BB, _NCAND, _KNN), 2)
    ic = jnp.broadcast_to(lax.shift_right_logical(fc_ref[...], 5)[..., None],
                          (_BB, _NCAND, 1))
    jc = jnp.broadcast_to(lax.bitwise_and(fc_ref[...], _KNN - 1)[..., None],
                          (_BB, _NCAND, 1))
    for h in range(_HEADS):
        q1 = q[:, h * _K_DIM: h * _K_DIM + _HALF]
        q2 = q[:, h * _K_DIM + _HALF: (h + 1) * _K_DIM]
        s1 = _bf16_dot(q1, k1_ref[h])
        s2 = _bf16_dot(q2, k2_ref[h])
        sc1, i1 = _topk32(s1)
        sc2, i2 = _topk32(s2)
        c1 = _pair_gather(sc1, ic, io3)
        c2 = _pair_gather(sc2, jc, io3)
        scores, code = _topk32_cand(c1 + c2 + cb, fcb)
        ipos = lax.shift_right_logical(code, 5)
        jpos = lax.bitwise_and(code, _KNN - 1)
        g1 = _gather_pos(i1, ipos)
        g2 = _gather_pos(i2, jpos)
        e = jnp.exp(scores - scores[:, 0:1])
        w = e / jnp.sum(e, axis=1, keepdims=True)
        idx_ref[:, h * _KNN:(h + 1) * _KNN] = g1 * _N_KEYS + g2
        w_ref[:, h * _KNN:(h + 1) * _KNN] = w


def _tc_call(x, W_q, keys1, keys2):
    nsel = _HEADS * _KNN
    nb = x.shape[0]
    return pl.pallas_call(
        _tc_body,
        grid=(nb // _BB,),
        in_specs=[
            pl.BlockSpec((_BB, _INPUT_DIM), lambda i: (i, 0)),
            pl.BlockSpec((_HEADS * _K_DIM, _INPUT_DIM), lambda i: (0, 0)),
            pl.BlockSpec((_HEADS, _N_KEYS, _HALF), lambda i: (0, 0, 0)),
            pl.BlockSpec((_HEADS, _N_KEYS, _HALF), lambda i: (0, 0, 0)),
            pl.BlockSpec((1, _NCAND), lambda i: (0, 0)),
            pl.BlockSpec((1, _NCAND), lambda i: (0, 0)),
        ],
        out_specs=[
            pl.BlockSpec((_BB, nsel), lambda i: (i, 0)),
            pl.BlockSpec((_BB, nsel), lambda i: (i, 0)),
        ],
        out_shape=[
            jax.ShapeDtypeStruct((nb, nsel), jnp.int32),
            jax.ShapeDtypeStruct((nb, nsel), jnp.float32),
        ],
    )(x, W_q, keys1, keys2, jnp.asarray(_FC), jnp.asarray(_CB))


def _sc_body(bpw, vals_hbm, idx_hbm, w_hbm, out_hbm, idx_v, w_v, rows_v,
             acc_v, sems):
    nch = 2 * bpw
    wid = lax.axis_index("s") * 2 + lax.axis_index("c")
    base = wid * bpw
    pltpu.sync_copy(idx_hbm.at[pl.ds(base, bpw)], idx_v)
    pltpu.sync_copy(w_hbm.at[pl.ds(base, bpw)], w_v)

    def copy_for(t, slot):
        b = lax.shift_right_logical(t, 1)
        half = lax.bitwise_and(t, 1)
        return pltpu.make_async_copy(
            vals_hbm.at[idx_v.at[b, pl.ds(half * 64, 64)]],
            rows_v.at[slot], sems.at[slot])

    copy_for(0, 0).start()
    nacc = _OUT_DIM // 16

    def t_body(t, acc):
        b = lax.shift_right_logical(t, 1)
        half = lax.bitwise_and(t, 1)
        slot = lax.bitwise_and(t, 1)
        zero = half == 0
        acc = tuple(jnp.where(zero, 0.0, a) for a in acc)

        @pl.when(t + 1 < nch)
        def _issue():
            copy_for(t + 1, lax.bitwise_and(t + 1, 1)).start()

        copy_for(t, slot).wait()

        def g_body(g, acc2):
            acc2 = list(acc2)
            wv16 = w_v[b, pl.ds(half * 64 + g * 16, 16)]
            for jj in range(16):
                wvec = jnp.full((16,), wv16[jj], jnp.float32)
                for c in range(nacc):
                    acc2[c] = acc2[c] + wvec * rows_v[
                        slot, g * 16 + jj, pl.ds(c * 16, 16)]
            return tuple(acc2)

        acc = lax.fori_loop(0, 4, g_body, acc)

        @pl.when(half == 1)
        def _flush():
            for c in range(nacc):
                acc_v[pl.ds(c * 16, 16)] = acc[c]
            pltpu.sync_copy(acc_v, out_hbm.at[base + b])

        return acc

    zeros = tuple(jnp.zeros((16,), jnp.float32) for _ in range(nacc))
    lax.fori_loop(0, nch, t_body, zeros)


def _sc_call(values, idx, w):
    import functools
    nsel = _HEADS * _KNN
    nb = idx.shape[0]
    bpw = nb // 32
    fn = pl.kernel(
        functools.partial(_sc_body, bpw),
        out_type=jax.ShapeDtypeStruct((nb, _OUT_DIM), jnp.float32),
        mesh=plsc.VectorSubcoreMesh(core_axis_name="c", subcore_axis_name="s",
                                    num_cores=2, num_subcores=16),
        scratch_types=[
            pltpu.VMEM((bpw, nsel), jnp.int32),
            pltpu.VMEM((bpw, nsel), jnp.float32),
            pltpu.VMEM((2, nsel // 2, _OUT_DIM), jnp.float32),
            pltpu.VMEM((_OUT_DIM,), jnp.float32),
            pltpu.SemaphoreType.DMA((2,)),
        ],
    )
    return fn(values, idx, w)


_NCHUNK = 2


def kernel(x, W_q, keys1, keys2, values):
    outs = []
    step = _B // _NCHUNK
    for c in range(_NCHUNK):
        xc = lax.slice_in_dim(x, c * step, (c + 1) * step, axis=0)
        idx, w = _tc_call(xc, W_q, keys1, keys2)
        outs.append(_sc_call(values, idx, w))
    return jnp.concatenate(outs, axis=0)


# speedup vs baseline: 3.8606x; 1.0931x over previous
import numpy as np

import jax
import jax.numpy as jnp
from jax import lax
from jax.experimental import pallas as pl
from jax.experimental.pallas import tpu as pltpu
from jax.experimental.pallas import tpu_sc as plsc

_B = 4096
_INPUT_DIM = 512
_OUT_DIM = 512
_K_DIM = 256
_HALF = _K_DIM // 2
_HEADS = 4
_KNN = 32
_N_KEYS = 512

_BB = 256
_NCAND = 128
_BPW = _B // 32

_NEGF = -3.0e38
_BIGI = 1 << 20

_HP = lax.Precision.HIGHEST


def _build_candidates():
    pairs = [(i, j) for i in range(_KNN) for j in range(_KNN)
             if (i + 1) * (j + 1) <= _KNN]
    fc = np.full((1, _NCAND), _BIGI, np.int32)
    cb = np.zeros((1, _NCAND), np.float32)
    for c, (i, j) in enumerate(pairs):
        fc[0, c] = i * _KNN + j
    cb[0, len(pairs):] = _NEGF
    return fc, cb


_FC, _CB = _build_candidates()


def _bf16_dot(a, b):
    return lax.dot_general(a.astype(jnp.bfloat16), b.astype(jnp.bfloat16),
                           (((1,), (1,)), ((), ())),
                           preferred_element_type=jnp.float32)


def _topk32(s):
    iota = lax.broadcasted_iota(jnp.int32, s.shape, 1)
    vals, idxs = [], []
    cur = s
    for _ in range(_KNN):
        m = jnp.max(cur, axis=1, keepdims=True)
        ix = jnp.min(jnp.where(cur == m, iota, _BIGI), axis=1, keepdims=True)
        vals.append(m)
        idxs.append(ix)
        cur = jnp.where(iota == ix, _NEGF, cur)
    return jnp.concatenate(vals, 1), jnp.concatenate(idxs, 1)


def _topk32_cand(cand, fcb):
    vals, codes = [], []
    cur = cand
    for _ in range(_KNN):
        m = jnp.max(cur, axis=1, keepdims=True)
        code = jnp.min(jnp.where(cur == m, fcb, _BIGI), axis=1, keepdims=True)
        vals.append(m)
        codes.append(code)
        cur = jnp.where(fcb == code, _NEGF, cur)
    return jnp.concatenate(vals, 1), jnp.concatenate(codes, 1)


def _gather_pos(tab, pos):
    ii = lax.broadcasted_iota(jnp.int32, (_BB, _KNN, _KNN), 2)
    onehot = pos[:, :, None] == ii
    return jnp.sum(jnp.where(onehot, tab[:, None, :], 0), axis=2)


def _pair_gather(sc, pos_const, io3):
    onehot = pos_const == io3
    return jnp.sum(jnp.where(onehot, sc[:, None, :], 0.0), axis=2)


def _tc_body(x_ref, wq_ref, k1_ref, k2_ref, fc_ref, cb_ref,
             idx_ref, w_ref):
    x = x_ref[...]
    q = _bf16_dot(x, wq_ref[...])
    fcb = jnp.broadcast_to(fc_ref[...], (_BB, _NCAND))
    cb = jnp.broadcast_to(cb_ref[...], (_BB, _NCAND))
    io3 = lax.broadcasted_iota(jnp.int32, (_BB, _NCAND, _KNN), 2)
    ic = jnp.broadcast_to(lax.shift_right_logical(fc_ref[...], 5)[..., None],
                          (_BB, _NCAND, 1))
    jc = jnp.broadcast_to(lax.bitwise_and(fc_ref[...], _KNN - 1)[..., None],
                          (_BB, _NCAND, 1))
    for h in range(_HEADS):
        q1 = q[:, h * _K_DIM: h * _K_DIM + _HALF]
        q2 = q[:, h * _K_DIM + _HALF: (h + 1) * _K_DIM]
        s1 = _bf16_dot(q1, k1_ref[h])
        s2 = _bf16_dot(q2, k2_ref[h])
        sc1, i1 = _topk32(s1)
        sc2, i2 = _topk32(s2)
        c1 = _pair_gather(sc1, ic, io3)
        c2 = _pair_gather(sc2, jc, io3)
        scores, code = _topk32_cand(c1 + c2 + cb, fcb)
        ipos = lax.shift_right_logical(code, 5)
        jpos = lax.bitwise_and(code, _KNN - 1)
        g1 = _gather_pos(i1, ipos)
        g2 = _gather_pos(i2, jpos)
        e = jnp.exp(scores - scores[:, 0:1])
        w = e / jnp.sum(e, axis=1, keepdims=True)
        idx_ref[:, h * _KNN:(h + 1) * _KNN] = g1 * _N_KEYS + g2
        w_ref[:, h * _KNN:(h + 1) * _KNN] = w


def _tc_call(x, W_q, keys1, keys2):
    nsel = _HEADS * _KNN
    nb = x.shape[0]
    return pl.pallas_call(
        _tc_body,
        grid=(nb // _BB,),
        in_specs=[
            pl.BlockSpec((_BB, _INPUT_DIM), lambda i: (i, 0)),
            pl.BlockSpec((_HEADS * _K_DIM, _INPUT_DIM), lambda i: (0, 0)),
            pl.BlockSpec((_HEADS, _N_KEYS, _HALF), lambda i: (0, 0, 0)),
            pl.BlockSpec((_HEADS, _N_KEYS, _HALF), lambda i: (0, 0, 0)),
            pl.BlockSpec((1, _NCAND), lambda i: (0, 0)),
            pl.BlockSpec((1, _NCAND), lambda i: (0, 0)),
        ],
        out_specs=[
            pl.BlockSpec((_BB, nsel), lambda i: (i, 0)),
            pl.BlockSpec((_BB, nsel), lambda i: (i, 0)),
        ],
        out_shape=[
            jax.ShapeDtypeStruct((nb, nsel), jnp.int32),
            jax.ShapeDtypeStruct((nb, nsel), jnp.float32),
        ],
    )(x, W_q, keys1, keys2, jnp.asarray(_FC), jnp.asarray(_CB))


def _sc_body(bpw, vals_hbm, idx_hbm, w_hbm, out_hbm, idx_v, w_v, rows_v,
             acc_v, sems):
    nch = 2 * bpw
    wid = lax.axis_index("s") * 2 + lax.axis_index("c")
    base = wid * bpw
    pltpu.sync_copy(idx_hbm.at[pl.ds(base, bpw)], idx_v)
    pltpu.sync_copy(w_hbm.at[pl.ds(base, bpw)], w_v)

    def copy_for(t, slot):
        b = lax.shift_right_logical(t, 1)
        half = lax.bitwise_and(t, 1)
        return pltpu.make_async_copy(
            vals_hbm.at[idx_v.at[b, pl.ds(half * 64, 64)]],
            rows_v.at[slot], sems.at[slot])

    copy_for(0, 0).start()
    nacc = _OUT_DIM // 16

    def t_body(t, acc):
        b = lax.shift_right_logical(t, 1)
        half = lax.bitwise_and(t, 1)
        slot = lax.bitwise_and(t, 1)
        zero = half == 0
        acc = tuple(jnp.where(zero, 0.0, a) for a in acc)

        @pl.when(t + 1 < nch)
        def _issue():
            copy_for(t + 1, lax.bitwise_and(t + 1, 1)).start()

        copy_for(t, slot).wait()

        def g_body(g, acc2):
            acc2 = list(acc2)
            wv16 = w_v[b, pl.ds(half * 64 + g * 16, 16)]
            for jj in range(16):
                wvec = jnp.full((16,), wv16[jj], jnp.float32)
                for c in range(nacc):
                    acc2[c] = acc2[c] + wvec * rows_v[
                        slot, g * 16 + jj, pl.ds(c * 16, 16)]
            return tuple(acc2)

        acc = lax.fori_loop(0, 4, g_body, acc)

        @pl.when(half == 1)
        def _flush():
            for c in range(nacc):
                acc_v[pl.ds(c * 16, 16)] = acc[c]
            pltpu.sync_copy(acc_v, out_hbm.at[base + b])

        return acc

    zeros = tuple(jnp.zeros((16,), jnp.float32) for _ in range(nacc))
    lax.fori_loop(0, nch, t_body, zeros)


def _sc_call(values, idx, w):
    import functools
    nsel = _HEADS * _KNN
    nb = idx.shape[0]
    bpw = nb // 32
    fn = pl.kernel(
        functools.partial(_sc_body, bpw),
        out_type=jax.ShapeDtypeStruct((nb, _OUT_DIM), jnp.float32),
        mesh=plsc.VectorSubcoreMesh(core_axis_name="c", subcore_axis_name="s",
                                    num_cores=2, num_subcores=16),
        scratch_types=[
            pltpu.VMEM((bpw, nsel), jnp.int32),
            pltpu.VMEM((bpw, nsel), jnp.float32),
            pltpu.VMEM((2, nsel // 2, _OUT_DIM), jnp.float32),
            pltpu.VMEM((_OUT_DIM,), jnp.float32),
            pltpu.SemaphoreType.DMA((2,)),
        ],
    )
    return fn(values, idx, w)


_NCHUNK = 4


def kernel(x, W_q, keys1, keys2, values):
    outs = []
    step = _B // _NCHUNK
    for c in range(_NCHUNK):
        xc = lax.slice_in_dim(x, c * step, (c + 1) * step, axis=0)
        idx, w = _tc_call(xc, W_q, keys1, keys2)
        outs.append(_sc_call(values, idx, w))
    return jnp.concatenate(outs, axis=0)


# speedup vs baseline: 7.1437x; 1.8504x over previous
import numpy as np

import jax
import jax.numpy as jnp
from jax import lax
from jax.experimental import pallas as pl
from jax.experimental.pallas import tpu as pltpu
from jax.experimental.pallas import tpu_sc as plsc

_B = 4096
_INPUT_DIM = 512
_OUT_DIM = 512
_K_DIM = 256
_HALF = _K_DIM // 2
_HEADS = 4
_KNN = 32
_N_KEYS = 512

_BB = 256
_NCAND = 128
_BPW = _B // 32

_NEGF = -3.0e38
_BIGI = 1 << 20

_HP = lax.Precision.HIGHEST


def _build_candidates():
    pairs = [(i, j) for i in range(_KNN) for j in range(_KNN)
             if (i + 1) * (j + 1) <= _KNN]
    fc = np.full((_NCAND, 1), _BIGI, np.int32)
    cb = np.zeros((_NCAND, 1), np.float32)
    for c, (i, j) in enumerate(pairs):
        fc[c, 0] = i * _KNN + j
    cb[len(pairs):, 0] = _NEGF
    return fc, cb


_FC, _CB = _build_candidates()


def _bf16_dot(a, b):
    return lax.dot_general(a.astype(jnp.bfloat16), b.astype(jnp.bfloat16),
                           (((1,), (1,)), ((), ())),
                           preferred_element_type=jnp.float32)


def _topk32_t(s):
    iota = lax.broadcasted_iota(jnp.int32, s.shape, 0)
    vals, idxs = [], []
    cur = s
    for _ in range(_KNN):
        m = jnp.max(cur, axis=0, keepdims=True)
        ix = jnp.min(jnp.where(cur == m, iota, _BIGI), axis=0, keepdims=True)
        vals.append(m)
        idxs.append(ix)
        cur = jnp.where(iota == ix, _NEGF, cur)
    return jnp.concatenate(vals, 0), jnp.concatenate(idxs, 0)


def _topk32_cand_t(cand, fcb):
    vals, codes = [], []
    cur = cand
    for _ in range(_KNN):
        m = jnp.max(cur, axis=0, keepdims=True)
        code = jnp.min(jnp.where(cur == m, fcb, _BIGI), axis=0, keepdims=True)
        vals.append(m)
        codes.append(code)
        cur = jnp.where(fcb == code, _NEGF, cur)
    return jnp.concatenate(vals, 0), jnp.concatenate(codes, 0)


def _gather_pos_t(tab, pos):
    ii = lax.broadcasted_iota(jnp.int32, (_KNN, _KNN, 1), 1)
    onehot = pos[:, None, :] == ii
    return jnp.sum(jnp.where(onehot, tab[None, :, :], 0), axis=1)


def _pair_gather_t(sc, pos_c, io3):
    onehot = pos_c == io3
    return jnp.sum(jnp.where(onehot, sc[None, :, :], 0.0), axis=1)


def _tc_body(x_ref, wq_ref, k1_ref, k2_ref, fc_ref, cb_ref,
             idx_ref, w_ref):
    x = x_ref[...]
    q = _bf16_dot(x, wq_ref[...])
    fc = fc_ref[...]
    fcb = jnp.broadcast_to(fc, (_NCAND, _BB))
    cb = jnp.broadcast_to(cb_ref[...], (_NCAND, _BB))
    io3 = lax.broadcasted_iota(jnp.int32, (_NCAND, _KNN, 1), 1)
    ic = lax.shift_right_logical(fc, 5)[:, :, None]
    jc = lax.bitwise_and(fc, _KNN - 1)[:, :, None]
    for h in range(_HEADS):
        q1 = q[:, h * _K_DIM: h * _K_DIM + _HALF]
        q2 = q[:, h * _K_DIM + _HALF: (h + 1) * _K_DIM]
        s1t = _bf16_dot(k1_ref[h], q1)
        s2t = _bf16_dot(k2_ref[h], q2)
        sc1, i1 = _topk32_t(s1t)
        sc2, i2 = _topk32_t(s2t)
        c1 = _pair_gather_t(sc1, ic, io3)
        c2 = _pair_gather_t(sc2, jc, io3)
        scores, code = _topk32_cand_t(c1 + c2 + cb, fcb)
        ipos = lax.shift_right_logical(code, 5)
        jpos = lax.bitwise_and(code, _KNN - 1)
        g1 = _gather_pos_t(i1, ipos)
        g2 = _gather_pos_t(i2, jpos)
        e = jnp.exp(scores - scores[0:1, :])
        w = e / jnp.sum(e, axis=0, keepdims=True)
        idx_ref[h * _KNN:(h + 1) * _KNN, :] = g1 * _N_KEYS + g2
        w_ref[h * _KNN:(h + 1) * _KNN, :] = w


def _tc_call(x, W_q, keys1, keys2):
    nsel = _HEADS * _KNN
    nb = x.shape[0]
    return pl.pallas_call(
        _tc_body,
        grid=(nb // _BB,),
        in_specs=[
            pl.BlockSpec((_BB, _INPUT_DIM), lambda i: (i, 0)),
            pl.BlockSpec((_HEADS * _K_DIM, _INPUT_DIM), lambda i: (0, 0)),
            pl.BlockSpec((_HEADS, _N_KEYS, _HALF), lambda i: (0, 0, 0)),
            pl.BlockSpec((_HEADS, _N_KEYS, _HALF), lambda i: (0, 0, 0)),
            pl.BlockSpec((_NCAND, 1), lambda i: (0, 0)),
            pl.BlockSpec((_NCAND, 1), lambda i: (0, 0)),
        ],
        out_specs=[
            pl.BlockSpec((nsel, _BB), lambda i: (0, i)),
            pl.BlockSpec((nsel, _BB), lambda i: (0, i)),
        ],
        out_shape=[
            jax.ShapeDtypeStruct((nsel, nb), jnp.int32),
            jax.ShapeDtypeStruct((nsel, nb), jnp.float32),
        ],
    )(x, W_q, keys1, keys2, jnp.asarray(_FC), jnp.asarray(_CB))


def _sc_body(bpw, vals_hbm, idx_hbm, w_hbm, out_hbm, idx_v, w_v, rows_v,
             acc_v, sems):
    nch = 2 * bpw
    wid = lax.axis_index("s") * 2 + lax.axis_index("c")
    base = wid * bpw
    pltpu.sync_copy(idx_hbm.at[pl.ds(base, bpw)], idx_v)
    pltpu.sync_copy(w_hbm.at[pl.ds(base, bpw)], w_v)

    def copy_for(t, slot):
        b = lax.shift_right_logical(t, 1)
        half = lax.bitwise_and(t, 1)
        return pltpu.make_async_copy(
            vals_hbm.at[idx_v.at[b, pl.ds(half * 64, 64)]],
            rows_v.at[slot], sems.at[slot])

    copy_for(0, 0).start()
    nacc = _OUT_DIM // 16

    def t_body(t, acc):
        b = lax.shift_right_logical(t, 1)
        half = lax.bitwise_and(t, 1)
        slot = lax.bitwise_and(t, 1)
        zero = half == 0
        acc = tuple(jnp.where(zero, 0.0, a) for a in acc)

        @pl.when(t + 1 < nch)
        def _issue():
            copy_for(t + 1, lax.bitwise_and(t + 1, 1)).start()

        copy_for(t, slot).wait()

        def g_body(g, acc2):
            acc2 = list(acc2)
            wv16 = w_v[b, pl.ds(half * 64 + g * 16, 16)]
            for jj in range(16):
                wvec = jnp.full((16,), wv16[jj], jnp.float32)
                for c in range(nacc):
                    acc2[c] = acc2[c] + wvec * rows_v[
                        slot, g * 16 + jj, pl.ds(c * 16, 16)]
            return tuple(acc2)

        acc = lax.fori_loop(0, 4, g_body, acc)

        @pl.when(half == 1)
        def _flush():
            for c in range(nacc):
                acc_v[pl.ds(c * 16, 16)] = acc[c]
            pltpu.sync_copy(acc_v, out_hbm.at[base + b])

        return acc

    zeros = tuple(jnp.zeros((16,), jnp.float32) for _ in range(nacc))
    lax.fori_loop(0, nch, t_body, zeros)


def _sc_call(values, idx, w):
    import functools
    nsel = _HEADS * _KNN
    nb = idx.shape[0]
    bpw = nb // 32
    fn = pl.kernel(
        functools.partial(_sc_body, bpw),
        out_type=jax.ShapeDtypeStruct((nb, _OUT_DIM), jnp.float32),
        mesh=plsc.VectorSubcoreMesh(core_axis_name="c", subcore_axis_name="s",
                                    num_cores=2, num_subcores=16),
        scratch_types=[
            pltpu.VMEM((bpw, nsel), jnp.int32),
            pltpu.VMEM((bpw, nsel), jnp.float32),
            pltpu.VMEM((2, nsel // 2, _OUT_DIM), jnp.float32),
            pltpu.VMEM((_OUT_DIM,), jnp.float32),
            pltpu.SemaphoreType.DMA((2,)),
        ],
    )
    return fn(values, idx, w)


_NCHUNK = 4


def kernel(x, W_q, keys1, keys2, values):
    outs = []
    step = _B // _NCHUNK
    for c in range(_NCHUNK):
        xc = lax.slice_in_dim(x, c * step, (c + 1) * step, axis=0)
        idx_t, w_t = _tc_call(xc, W_q, keys1, keys2)
        outs.append(_sc_call(values, idx_t.T, w_t.T))
    return jnp.concatenate(outs, axis=0)


# speedup vs baseline: 8.0632x; 1.1287x over previous
import numpy as np

import jax
import jax.numpy as jnp
from jax import lax
from jax.experimental import pallas as pl
from jax.experimental.pallas import tpu as pltpu
from jax.experimental.pallas import tpu_sc as plsc

_B = 4096
_INPUT_DIM = 512
_OUT_DIM = 512
_K_DIM = 256
_HALF = _K_DIM // 2
_HEADS = 4
_KNN = 32
_N_KEYS = 512

_BB = 256
_NCAND = 128
_BPW = _B // 32

_NEGF = -3.0e38
_BIGI = 1 << 20

_HP = lax.Precision.HIGHEST


def _build_candidates():
    pairs = [(i, j) for i in range(_KNN) for j in range(_KNN)
             if (i + 1) * (j + 1) <= _KNN]
    fc = np.full((_NCAND, 1), _BIGI, np.int32)
    cb = np.zeros((_NCAND, 1), np.float32)
    for c, (i, j) in enumerate(pairs):
        fc[c, 0] = i * _KNN + j
    cb[len(pairs):, 0] = _NEGF
    return fc, cb


_FC, _CB = _build_candidates()


def _bf16_dot(a, b):
    return lax.dot_general(a.astype(jnp.bfloat16), b.astype(jnp.bfloat16),
                           (((1,), (1,)), ((), ())),
                           preferred_element_type=jnp.float32)


def _topk32_t(s):
    iota = lax.broadcasted_iota(jnp.int32, s.shape, 0)
    vals, idxs = [], []
    cur = s
    for _ in range(_KNN):
        m = jnp.max(cur, axis=0, keepdims=True)
        ix = jnp.min(jnp.where(cur == m, iota, _BIGI), axis=0, keepdims=True)
        vals.append(m)
        idxs.append(ix)
        cur = jnp.where(iota == ix, _NEGF, cur)
    return jnp.concatenate(vals, 0), jnp.concatenate(idxs, 0)


def _topk32_cand_t(cand, fcb):
    vals, codes = [], []
    cur = cand
    for _ in range(_KNN):
        m = jnp.max(cur, axis=0, keepdims=True)
        code = jnp.min(jnp.where(cur == m, fcb, _BIGI), axis=0, keepdims=True)
        vals.append(m)
        codes.append(code)
        cur = jnp.where(fcb == code, _NEGF, cur)
    return jnp.concatenate(vals, 0), jnp.concatenate(codes, 0)


def _gather_pos_t(tab, pos):
    ii = lax.broadcasted_iota(jnp.int32, (_KNN, _KNN, 1), 1)
    onehot = pos[:, None, :] == ii
    return jnp.sum(jnp.where(onehot, tab[None, :, :], 0), axis=1)


def _pair_gather_t(sc, pos_c, io3):
    onehot = pos_c == io3
    return jnp.sum(jnp.where(onehot, sc[None, :, :], 0.0), axis=1)


def _tc_body(x_ref, wq_ref, k1_ref, k2_ref, fc_ref, cb_ref,
             idx_ref, w_ref):
    x = x_ref[...]
    q = _bf16_dot(x, wq_ref[...])
    fc = fc_ref[...]
    fcb = jnp.broadcast_to(fc, (_NCAND, _BB))
    cb = jnp.broadcast_to(cb_ref[...], (_NCAND, _BB))
    io3 = lax.broadcasted_iota(jnp.int32, (_NCAND, _KNN, 1), 1)
    ic = lax.shift_right_logical(fc, 5)[:, :, None]
    jc = lax.bitwise_and(fc, _KNN - 1)[:, :, None]
    for h in range(_HEADS):
        q1 = q[:, h * _K_DIM: h * _K_DIM + _HALF]
        q2 = q[:, h * _K_DIM + _HALF: (h + 1) * _K_DIM]
        s1t = _bf16_dot(k1_ref[h], q1)
        s2t = _bf16_dot(k2_ref[h], q2)
        sc1, i1 = _topk32_t(s1t)
        sc2, i2 = _topk32_t(s2t)
        c1 = _pair_gather_t(sc1, ic, io3)
        c2 = _pair_gather_t(sc2, jc, io3)
        scores, code = _topk32_cand_t(c1 + c2 + cb, fcb)
        ipos = lax.shift_right_logical(code, 5)
        jpos = lax.bitwise_and(code, _KNN - 1)
        g1 = _gather_pos_t(i1, ipos)
        g2 = _gather_pos_t(i2, jpos)
        e = jnp.exp(scores - scores[0:1, :])
        w = e / jnp.sum(e, axis=0, keepdims=True)
        idx_ref[h * _KNN:(h + 1) * _KNN, :] = g1 * _N_KEYS + g2
        w_ref[h * _KNN:(h + 1) * _KNN, :] = w


def _tc_call(x, W_q, keys1, keys2):
    nsel = _HEADS * _KNN
    nb = x.shape[0]
    return pl.pallas_call(
        _tc_body,
        grid=(nb // _BB,),
        in_specs=[
            pl.BlockSpec((_BB, _INPUT_DIM), lambda i: (i, 0)),
            pl.BlockSpec((_HEADS * _K_DIM, _INPUT_DIM), lambda i: (0, 0)),
            pl.BlockSpec((_HEADS, _N_KEYS, _HALF), lambda i: (0, 0, 0)),
            pl.BlockSpec((_HEADS, _N_KEYS, _HALF), lambda i: (0, 0, 0)),
            pl.BlockSpec((_NCAND, 1), lambda i: (0, 0)),
            pl.BlockSpec((_NCAND, 1), lambda i: (0, 0)),
        ],
        out_specs=[
            pl.BlockSpec((nsel, _BB), lambda i: (0, i)),
            pl.BlockSpec((nsel, _BB), lambda i: (0, i)),
        ],
        out_shape=[
            jax.ShapeDtypeStruct((nsel, nb), jnp.int32),
            jax.ShapeDtypeStruct((nsel, nb), jnp.float32),
        ],
    )(x, W_q, keys1, keys2, jnp.asarray(_FC), jnp.asarray(_CB))


def _sc_body(bpw, vals_hbm, idx_hbm, w_hbm, out_hbm, idx_v, w_v, rows_v,
             acc_v, sems):
    nch = 2 * bpw
    wid = lax.axis_index("s") * 2 + lax.axis_index("c")
    base = wid * bpw
    pltpu.sync_copy(idx_hbm.at[pl.ds(base, bpw)], idx_v)
    pltpu.sync_copy(w_hbm.at[pl.ds(base, bpw)], w_v)

    def copy_for(t, slot):
        b = lax.shift_right_logical(t, 1)
        half = lax.bitwise_and(t, 1)
        return pltpu.make_async_copy(
            vals_hbm.at[idx_v.at[b, pl.ds(half * 64, 64)]],
            rows_v.at[slot], sems.at[slot])

    copy_for(0, 0).start()
    copy_for(1, 1).start()
    nacc = _OUT_DIM // 16

    def t_body(t, acc):
        b = lax.shift_right_logical(t, 1)
        half = lax.bitwise_and(t, 1)
        slot = lax.rem(t, 3)
        zero = half == 0
        acc = tuple(jnp.where(zero, 0.0, a) for a in acc)

        @pl.when(t + 2 < nch)
        def _issue():
            copy_for(t + 2, lax.rem(t + 2, 3)).start()

        copy_for(t, slot).wait()

        def g_body(g, acc2):
            acc2 = list(acc2)
            wv16 = w_v[b, pl.ds(half * 64 + g * 16, 16)]
            for jj in range(16):
                wvec = jnp.full((16,), wv16[jj], jnp.float32)
                for c in range(nacc):
                    acc2[c] = acc2[c] + wvec * rows_v[
                        slot, g * 16 + jj, pl.ds(c * 16, 16)]
            return tuple(acc2)

        acc = lax.fori_loop(0, 4, g_body, acc)

        @pl.when(half == 1)
        def _flush():
            for c in range(nacc):
                acc_v[pl.ds(c * 16, 16)] = acc[c]
            pltpu.sync_copy(acc_v, out_hbm.at[base + b])

        return acc

    zeros = tuple(jnp.zeros((16,), jnp.float32) for _ in range(nacc))
    lax.fori_loop(0, nch, t_body, zeros)


def _sc_call(values, idx, w):
    import functools
    nsel = _HEADS * _KNN
    nb = idx.shape[0]
    bpw = nb // 32
    fn = pl.kernel(
        functools.partial(_sc_body, bpw),
        out_type=jax.ShapeDtypeStruct((nb, _OUT_DIM), jnp.float32),
        mesh=plsc.VectorSubcoreMesh(core_axis_name="c", subcore_axis_name="s",
                                    num_cores=2, num_subcores=16),
        scratch_types=[
            pltpu.VMEM((bpw, nsel), jnp.int32),
            pltpu.VMEM((bpw, nsel), jnp.float32),
            pltpu.VMEM((3, nsel // 2, _OUT_DIM), jnp.float32),
            pltpu.VMEM((_OUT_DIM,), jnp.float32),
            pltpu.SemaphoreType.DMA((3,)),
        ],
    )
    return fn(values, idx, w)


_NCHUNK = 4


def kernel(x, W_q, keys1, keys2, values):
    outs = []
    step = _B // _NCHUNK
    for c in range(_NCHUNK):
        xc = lax.slice_in_dim(x, c * step, (c + 1) * step, axis=0)
        idx_t, w_t = _tc_call(xc, W_q, keys1, keys2)
        outs.append(_sc_call(values, idx_t.T, w_t.T))
    return jnp.concatenate(outs, axis=0)
